# Initial kernel scaffold; baseline (speedup 1.0000x reference)
import jax
import jax.numpy as jnp
from jax.experimental import pallas as pl


def kernel(entity_ids, relation_ids, entity_cluster_labels, relation_cluster_labels, entity_table, relation_table):
    raise NotImplementedError("write your pallas kernel here")


# speedup vs baseline: 4.9091x; 4.9091x over previous
import functools

import jax
import jax.numpy as jnp
from jax import lax
from jax.experimental import pallas as pl
from jax.experimental.pallas import tpu as pltpu
from jax.experimental.pallas import tpu_sc as plsc

_NUM_ENTITIES = 256
_NUM_RELATIONS = 64
_EMBED_DIM = 128
_BATCH = 16384
_HIST = 20

_NC = 2
_NS = 16
_L = 16
_NW = _NC * _NS

_B_E = _BATCH * _HIST
_B_R = _BATCH
_EPW = _B_E // _NW
_RPW = _B_R // _NW
_CH = 128
_E_CHUNKS = _EPW // _CH
_R_CHUNKS = _RPW // _CH

_mesh = plsc.VectorSubcoreMesh(core_axis_name="c", subcore_axis_name="s")


@functools.partial(
    pl.kernel,
    mesh=_mesh,
    compiler_params=pltpu.CompilerParams(needs_layout_passes=False),
    out_type=(
        jax.ShapeDtypeStruct((_B_E, _EMBED_DIM), jnp.float32),
        jax.ShapeDtypeStruct((_B_R, _EMBED_DIM), jnp.float32),
    ),
    scratch_types=[
        pltpu.VMEM((_EPW,), jnp.int32),
        pltpu.VMEM((_RPW,), jnp.int32),
        pltpu.VMEM((_NUM_ENTITIES,), jnp.int32),
        pltpu.VMEM((_NUM_RELATIONS,), jnp.int32),
        pltpu.VMEM((_CH,), jnp.int32),
        pltpu.VMEM((_CH, _EMBED_DIM), jnp.float32),
        pltpu.SemaphoreType.DMA,
    ],
)
def _cluster_embed_sc(eids, rids, elab, rlab, etab, rtab, out_e, out_r,
                      ids_v, rids_v, elab_v, rlab_v, cl_v, rows_v, sem):
    wid = lax.axis_index("s") * _NC + lax.axis_index("c")
    ebase = wid * _EPW
    rbase = wid * _RPW

    pltpu.sync_copy(eids.at[pl.ds(ebase, _EPW)], ids_v)
    pltpu.sync_copy(rids.at[pl.ds(rbase, _RPW)], rids_v)
    pltpu.sync_copy(elab, elab_v)
    pltpu.sync_copy(rlab, rlab_v)

    def _echunk(j, carry):
        for k in range(_CH // _L):
            v = ids_v[pl.ds(j * _CH + k * _L, _L)]
            cl_v[pl.ds(k * _L, _L)] = plsc.load_gather(elab_v, [v])
        pltpu.async_copy(etab.at[cl_v], rows_v, sem).wait()
        pltpu.sync_copy(rows_v, out_e.at[pl.ds(ebase + j * _CH, _CH)])
        return carry

    lax.fori_loop(0, _E_CHUNKS, _echunk, 0)

    def _rchunk(j, carry):
        for k in range(_CH // _L):
            v = rids_v[pl.ds(j * _CH + k * _L, _L)]
            cl_v[pl.ds(k * _L, _L)] = plsc.load_gather(rlab_v, [v])
        pltpu.async_copy(rtab.at[cl_v], rows_v, sem).wait()
        pltpu.sync_copy(rows_v, out_r.at[pl.ds(rbase + j * _CH, _CH)])
        return carry

    lax.fori_loop(0, _R_CHUNKS, _rchunk, 0)


def kernel(entity_ids, relation_ids, entity_cluster_labels,
           relation_cluster_labels, entity_table, relation_table):
    ent_flat, rel_emb = _cluster_embed_sc(
        entity_ids.reshape(-1),
        relation_ids,
        entity_cluster_labels,
        relation_cluster_labels,
        entity_table,
        relation_table,
    )
    return ent_flat.reshape(_BATCH, _HIST, _EMBED_DIM), rel_emb


# speedup vs baseline: 4.9811x; 1.0147x over previous
import functools

import jax
import jax.numpy as jnp
from jax import lax
from jax.experimental import pallas as pl
from jax.experimental.pallas import tpu as pltpu
from jax.experimental.pallas import tpu_sc as plsc

_NUM_ENTITIES = 256
_NUM_RELATIONS = 64
_EMBED_DIM = 128
_BATCH = 16384
_HIST = 20

_NC = 2
_NS = 16
_L = 16
_NW = _NC * _NS

_B_E = _BATCH * _HIST
_B_R = _BATCH
_EPW = _B_E // _NW
_RPW = _B_R // _NW
_CH = 128
_E_CHUNKS = _EPW // _CH
_R_CHUNKS = _RPW // _CH
_NBUF = 4
_E_GROUPS = _E_CHUNKS // _NBUF

_mesh = plsc.VectorSubcoreMesh(core_axis_name="c", subcore_axis_name="s")


@functools.partial(
    pl.kernel,
    mesh=_mesh,
    compiler_params=pltpu.CompilerParams(needs_layout_passes=False),
    out_type=(
        jax.ShapeDtypeStruct((_B_E, _EMBED_DIM), jnp.float32),
        jax.ShapeDtypeStruct((_B_R, _EMBED_DIM), jnp.float32),
    ),
    scratch_types=[
        pltpu.VMEM((_EPW,), jnp.int32),
        pltpu.VMEM((_RPW,), jnp.int32),
        pltpu.VMEM((_NUM_ENTITIES,), jnp.int32),
        pltpu.VMEM((_NUM_RELATIONS,), jnp.int32),
        [pltpu.VMEM((_CH,), jnp.int32) for _ in range(_NBUF)],
        [pltpu.VMEM((_CH, _EMBED_DIM), jnp.float32) for _ in range(_NBUF)],
        [pltpu.SemaphoreType.DMA for _ in range(_NBUF)],
        [pltpu.SemaphoreType.DMA for _ in range(_NBUF)],
    ],
)
def _cluster_embed_sc(eids, rids, elab, rlab, etab, rtab, out_e, out_r,
                      ids_v, rids_v, elab_v, rlab_v, cl_v, rows_v,
                      g_sem, w_sem):
    wid = lax.axis_index("s") * _NC + lax.axis_index("c")
    ebase = wid * _EPW
    rbase = wid * _RPW

    pltpu.sync_copy(eids.at[pl.ds(ebase, _EPW)], ids_v)
    pltpu.sync_copy(rids.at[pl.ds(rbase, _RPW)], rids_v)
    pltpu.sync_copy(elab, elab_v)
    pltpu.sync_copy(rlab, rlab_v)

    def _compose(ids_ref, lab_ref, cl_ref, chunk):
        for k in range(_CH // _L):
            v = ids_ref[pl.ds(chunk * _CH + k * _L, _L)]
            cl_ref[pl.ds(k * _L, _L)] = plsc.load_gather(lab_ref, [v])

    for b in range(_NBUF):
        _compose(ids_v, elab_v, cl_v[b], b)
        pltpu.async_copy(etab.at[cl_v[b]], rows_v[b], g_sem[b])

    def _egroup(gi, carry):
        g = gi * _NBUF
        for b in range(_NBUF):
            pltpu.make_async_copy(etab.at[cl_v[b]], rows_v[b], g_sem[b]).wait()
            pltpu.async_copy(rows_v[b], out_e.at[pl.ds(ebase + (g + b) * _CH, _CH)],
                             w_sem[b])
        for b in range(_NBUF):
            @pl.when(gi + 1 < _E_GROUPS)
            def _():
                _compose(ids_v, elab_v, cl_v[b], g + _NBUF + b)
        for b in range(_NBUF):
            pltpu.make_async_copy(
                rows_v[b], out_e.at[pl.ds(ebase, _CH)], w_sem[b]).wait()
            @pl.when(gi + 1 < _E_GROUPS)
            def _():
                pltpu.async_copy(etab.at[cl_v[b]], rows_v[b], g_sem[b])
        return carry

    lax.fori_loop(0, _E_GROUPS, _egroup, 0)

    for b in range(_R_CHUNKS):
        bb = b % _NBUF
        _compose(rids_v, rlab_v, cl_v[bb], b)
        pltpu.async_copy(rtab.at[cl_v[bb]], rows_v[bb], g_sem[bb])
    for b in range(_R_CHUNKS):
        bb = b % _NBUF
        pltpu.make_async_copy(rtab.at[cl_v[bb]], rows_v[bb], g_sem[bb]).wait()
        pltpu.async_copy(rows_v[bb], out_r.at[pl.ds(rbase + b * _CH, _CH)],
                         w_sem[bb])
    for b in range(_R_CHUNKS):
        bb = b % _NBUF
        pltpu.make_async_copy(
            rows_v[bb], out_r.at[pl.ds(rbase, _CH)], w_sem[bb]).wait()


def kernel(entity_ids, relation_ids, entity_cluster_labels,
           relation_cluster_labels, entity_table, relation_table):
    ent_flat, rel_emb = _cluster_embed_sc(
        entity_ids.reshape(-1),
        relation_ids,
        entity_cluster_labels,
        relation_cluster_labels,
        entity_table,
        relation_table,
    )
    return ent_flat.reshape(_BATCH, _HIST, _EMBED_DIM), rel_emb


# speedup vs baseline: 7.7604x; 1.5580x over previous
import functools

import jax
import jax.numpy as jnp
from jax import lax
from jax.experimental import pallas as pl
from jax.experimental.pallas import tpu as pltpu
from jax.experimental.pallas import tpu_sc as plsc

_NUM_ENTITIES = 256
_NUM_RELATIONS = 64
_EMBED_DIM = 128
_BATCH = 16384
_HIST = 20

_NC = 2
_NS = 16
_L = 16
_NW = _NC * _NS

_B_E = _BATCH * _HIST
_EPW = _B_E // _NW
_BPW = _BATCH // _NW
_RPW = _BATCH // _NW
_WB = 4
_CL = _WB * _HIST
_E_CHUNKS = _BPW // _WB
_E_GROUPS = _E_CHUNKS // 2
_RCH = 128
_R_CHUNKS = _RPW // _RCH

_mesh = plsc.VectorSubcoreMesh(core_axis_name="c", subcore_axis_name="s")


@functools.partial(
    pl.kernel,
    mesh=_mesh,
    compiler_params=pltpu.CompilerParams(needs_layout_passes=False),
    out_type=(
        jax.ShapeDtypeStruct((_BATCH, _HIST, _EMBED_DIM), jnp.float32),
        jax.ShapeDtypeStruct((_BATCH, _EMBED_DIM), jnp.float32),
    ),
    scratch_types=[
        pltpu.VMEM((_EPW,), jnp.int32),
        pltpu.VMEM((_RPW,), jnp.int32),
        pltpu.VMEM((_NUM_ENTITIES,), jnp.int32),
        pltpu.VMEM((_NUM_RELATIONS,), jnp.int32),
        pltpu.VMEM((64 * _EMBED_DIM,), jnp.float32),
        [pltpu.VMEM((_CL, _EMBED_DIM), jnp.float32) for _ in range(2)],
        [pltpu.VMEM((_RCH,), jnp.int32) for _ in range(_R_CHUNKS)],
        [pltpu.VMEM((_RCH, _EMBED_DIM), jnp.float32) for _ in range(_R_CHUNKS)],
        [pltpu.SemaphoreType.DMA for _ in range(2)],
        [pltpu.SemaphoreType.DMA for _ in range(_R_CHUNKS)],
    ],
)
def _cluster_embed_sc(eids, rids, elab, rlab, etab_flat, rtab, out_e, out_r,
                      ids_v, rids_v, elab_v, rlab_v, etab_v,
                      rows3_v, rcl_v, rbuf_v, w_sem, r_sem):
    wid = lax.axis_index("s") * _NC + lax.axis_index("c")
    ebase = wid * _EPW
    bbase = wid * _BPW
    rbase = wid * _RPW

    pltpu.sync_copy(eids.at[pl.ds(ebase, _EPW)], ids_v)
    pltpu.sync_copy(rids.at[pl.ds(rbase, _RPW)], rids_v)
    pltpu.sync_copy(elab, elab_v)
    pltpu.sync_copy(rlab, rlab_v)
    pltpu.sync_copy(etab_flat, etab_v)

    for c in range(_R_CHUNKS):
        for k in range(_RCH // _L):
            v = rids_v[pl.ds(c * _RCH + k * _L, _L)]
            rcl_v[c][pl.ds(k * _L, _L)] = plsc.load_gather(rlab_v, [v])
        pltpu.async_copy(rtab.at[rcl_v[c]], rbuf_v[c], r_sem[c])

    def _egroup(g, carry):
        for b in range(2):
            c = g * 2 + b

            @pl.when(c >= 2)
            def _():
                for i in range(_WB):
                    pltpu.make_async_copy(
                        rows3_v[b].at[pl.ds(i * _HIST, _HIST)],
                        out_e.at[bbase + i], w_sem[b]).wait()

            for grp in range(_CL // _L):
                ids16 = ids_v[pl.ds(c * _CL + grp * _L, _L)]
                cl16 = plsc.load_gather(elab_v, [ids16])
                for l in range(_L):
                    r = grp * _L + l
                    base = cl16[l] * _EMBED_DIM
                    for k in range(_EMBED_DIM // _L):
                        rows3_v[b][r, pl.ds(k * _L, _L)] = (
                            etab_v[pl.ds(base + k * _L, _L)])

            for i in range(_WB):
                pltpu.async_copy(
                    rows3_v[b].at[pl.ds(i * _HIST, _HIST)],
                    out_e.at[bbase + c * _WB + i], w_sem[b])
        return carry

    lax.fori_loop(0, _E_GROUPS, _egroup, 0)
    for b in range(2):
        for i in range(_WB):
            pltpu.make_async_copy(
                rows3_v[b].at[pl.ds(i * _HIST, _HIST)],
                out_e.at[bbase + i], w_sem[b]).wait()

    for c in range(_R_CHUNKS):
        pltpu.make_async_copy(rtab.at[rcl_v[c]], rbuf_v[c], r_sem[c]).wait()
        pltpu.async_copy(
            rbuf_v[c], out_r.at[pl.ds(rbase + c * _RCH, _RCH)], r_sem[c])
    for c in range(_R_CHUNKS):
        pltpu.make_async_copy(
            rbuf_v[c], out_r.at[pl.ds(rbase, _RCH)], r_sem[c]).wait()


def kernel(entity_ids, relation_ids, entity_cluster_labels,
           relation_cluster_labels, entity_table, relation_table):
    return _cluster_embed_sc(
        entity_ids.reshape(-1),
        relation_ids,
        entity_cluster_labels,
        relation_cluster_labels,
        entity_table.reshape(-1),
        relation_table,
    )


# speedup vs baseline: 9.2740x; 1.1950x over previous
import functools

import jax
import jax.numpy as jnp
from jax import lax
from jax.experimental import pallas as pl
from jax.experimental.pallas import tpu as pltpu
from jax.experimental.pallas import tpu_sc as plsc

_NUM_ENTITIES = 256
_NUM_RELATIONS = 64
_EMBED_DIM = 128
_BATCH = 16384
_HIST = 20

_NC = 2
_NS = 16
_L = 16
_NW = _NC * _NS

_B_E = _BATCH * _HIST
_EPW = _B_E // _NW
_BPW = _BATCH // _NW
_RPW = _BATCH // _NW
_WB = 4
_CL = _WB * _HIST
_E_CHUNKS = _BPW // _WB
_E_GROUPS = _E_CHUNKS // 2
_RCH = 128
_R_CHUNKS = _RPW // _RCH

_mesh = plsc.VectorSubcoreMesh(core_axis_name="c", subcore_axis_name="s")


@functools.partial(
    pl.kernel,
    mesh=_mesh,
    compiler_params=pltpu.CompilerParams(needs_layout_passes=False),
    out_type=(
        jax.ShapeDtypeStruct((_BATCH, _HIST, _EMBED_DIM), jnp.float32),
        jax.ShapeDtypeStruct((_BATCH, _EMBED_DIM), jnp.float32),
    ),
    scratch_types=[
        pltpu.VMEM((_EPW,), jnp.int32),
        pltpu.VMEM((_RPW,), jnp.int32),
        pltpu.VMEM((_NUM_ENTITIES,), jnp.int32),
        pltpu.VMEM((_NUM_RELATIONS,), jnp.int32),
        pltpu.VMEM((64 * _EMBED_DIM,), jnp.float32),
        [pltpu.VMEM((_CL, _EMBED_DIM), jnp.float32) for _ in range(2)],
        [pltpu.VMEM((_RCH,), jnp.int32) for _ in range(_R_CHUNKS)],
        [pltpu.VMEM((_RCH, _EMBED_DIM), jnp.float32) for _ in range(_R_CHUNKS)],
        [pltpu.SemaphoreType.DMA for _ in range(2)],
        [pltpu.SemaphoreType.DMA for _ in range(_R_CHUNKS)],
    ],
)
def _cluster_embed_sc(eids, rids, elab, rlab, etab_flat, rtab, out_e, out_r,
                      ids_v, rids_v, elab_v, rlab_v, etab_v,
                      rows3_v, rcl_v, rbuf_v, w_sem, r_sem):
    wid = lax.axis_index("s") * _NC + lax.axis_index("c")
    ebase = wid * _EPW
    bbase = wid * _BPW
    rbase = wid * _RPW

    pltpu.sync_copy(eids.at[pl.ds(ebase, _EPW)], ids_v)
    pltpu.sync_copy(rids.at[pl.ds(rbase, _RPW)], rids_v)
    pltpu.sync_copy(elab, elab_v)
    pltpu.sync_copy(rlab, rlab_v)
    pltpu.sync_copy(etab_flat, etab_v)

    for c in range(_R_CHUNKS):
        for k in range(_RCH // _L):
            v = rids_v[pl.ds(c * _RCH + k * _L, _L)]
            rcl_v[c][pl.ds(k * _L, _L)] = plsc.load_gather(rlab_v, [v])
        pltpu.async_copy(rtab.at[rcl_v[c]], rbuf_v[c], r_sem[c])

    def _egroup(g, carry):
        for b in range(2):
            c = g * 2 + b

            @pl.when(c >= 2)
            def _():
                for i in range(_WB):
                    pltpu.make_async_copy(
                        rows3_v[b].at[pl.ds(i * _HIST, _HIST)],
                        out_e.at[bbase + i], w_sem[b]).wait()

            for grp in range(_CL // _L):
                ids16 = ids_v[pl.ds(c * _CL + grp * _L, _L)]
                cl16 = plsc.load_gather(elab_v, [ids16])
                bases = [cl16[l] * _EMBED_DIM for l in range(_L)]
                for k in range(_EMBED_DIM // _L):
                    vals = [etab_v[pl.ds(bases[l] + k * _L, _L)]
                            for l in range(_L)]
                    for l in range(_L):
                        rows3_v[b][grp * _L + l, pl.ds(k * _L, _L)] = vals[l]

            for i in range(_WB):
                pltpu.async_copy(
                    rows3_v[b].at[pl.ds(i * _HIST, _HIST)],
                    out_e.at[bbase + c * _WB + i], w_sem[b])
        return carry

    lax.fori_loop(0, _E_GROUPS, _egroup, 0)
    for b in range(2):
        for i in range(_WB):
            pltpu.make_async_copy(
                rows3_v[b].at[pl.ds(i * _HIST, _HIST)],
                out_e.at[bbase + i], w_sem[b]).wait()

    for c in range(_R_CHUNKS):
        pltpu.make_async_copy(rtab.at[rcl_v[c]], rbuf_v[c], r_sem[c]).wait()
        pltpu.async_copy(
            rbuf_v[c], out_r.at[pl.ds(rbase + c * _RCH, _RCH)], r_sem[c])
    for c in range(_R_CHUNKS):
        pltpu.make_async_copy(
            rbuf_v[c], out_r.at[pl.ds(rbase, _RCH)], r_sem[c]).wait()


def kernel(entity_ids, relation_ids, entity_cluster_labels,
           relation_cluster_labels, entity_table, relation_table):
    return _cluster_embed_sc(
        entity_ids.reshape(-1),
        relation_ids,
        entity_cluster_labels,
        relation_cluster_labels,
        entity_table.reshape(-1),
        relation_table,
    )


# speedup vs baseline: 11.6383x; 1.2549x over previous
import functools

import jax
import jax.numpy as jnp
from jax import lax
from jax.experimental import pallas as pl
from jax.experimental.pallas import tpu as pltpu
from jax.experimental.pallas import tpu_sc as plsc

_NUM_ENTITIES = 256
_NUM_RELATIONS = 64
_EMBED_DIM = 128
_BATCH = 16384
_HIST = 20

_NC = 2
_NS = 16
_L = 16
_NW = _NC * _NS

_B_E = _BATCH * _HIST
_EPW = _B_E // _NW
_BPW = _BATCH // _NW
_RPW = _BATCH // _NW
_WB = 4
_CL = _WB * _HIST
_E_CHUNKS = _BPW // _WB
_E_GROUPS = _E_CHUNKS // 2
_RCH = 128
_R_CHUNKS = _RPW // _RCH

_mesh = plsc.VectorSubcoreMesh(core_axis_name="c", subcore_axis_name="s")


@functools.partial(
    pl.kernel,
    mesh=_mesh,
    compiler_params=pltpu.CompilerParams(needs_layout_passes=False),
    out_type=(
        jax.ShapeDtypeStruct((_BATCH, _HIST, _EMBED_DIM), jnp.float32),
        jax.ShapeDtypeStruct((_BATCH, _EMBED_DIM), jnp.float32),
    ),
    scratch_types=[
        pltpu.VMEM((_EPW,), jnp.int32),
        pltpu.VMEM((_RPW,), jnp.int32),
        pltpu.VMEM((_NUM_ENTITIES,), jnp.int32),
        pltpu.VMEM((_NUM_RELATIONS,), jnp.int32),
        pltpu.VMEM((64 * _EMBED_DIM,), jnp.float32),
        [pltpu.VMEM((_CL, _EMBED_DIM), jnp.float32) for _ in range(2)],
        [pltpu.VMEM((_RCH,), jnp.int32) for _ in range(_R_CHUNKS)],
        [pltpu.VMEM((_RCH, _EMBED_DIM), jnp.float32) for _ in range(_R_CHUNKS)],
        [pltpu.SemaphoreType.DMA for _ in range(2)],
        [pltpu.SemaphoreType.DMA for _ in range(_R_CHUNKS)],
    ],
)
def _cluster_embed_sc(eids, rids, elab, rlab, etab_flat, rtab, out_e, out_r,
                      ids_v, rids_v, elab_v, rlab_v, etab_v,
                      rows3_v, rcl_v, rbuf_v, w_sem, r_sem):
    wid = lax.axis_index("s") * _NC + lax.axis_index("c")
    ebase = wid * _EPW
    bbase = wid * _BPW
    rbase = wid * _RPW

    pltpu.sync_copy(eids.at[pl.ds(ebase, _EPW)], ids_v)
    pltpu.sync_copy(rids.at[pl.ds(rbase, _RPW)], rids_v)
    pltpu.sync_copy(elab, elab_v)
    pltpu.sync_copy(rlab, rlab_v)
    pltpu.sync_copy(etab_flat, etab_v)

    for c in range(_R_CHUNKS):
        for k in range(_RCH // _L):
            v = rids_v[pl.ds(c * _RCH + k * _L, _L)]
            rcl_v[c][pl.ds(k * _L, _L)] = plsc.load_gather(rlab_v, [v])
        pltpu.async_copy(rtab.at[rcl_v[c]], rbuf_v[c], r_sem[c])

    def _egroup(g, carry):
        for b in range(2):
            c = g * 2 + b

            @pl.when(c >= 2)
            def _():
                for i in range(_WB):
                    pltpu.make_async_copy(
                        rows3_v[b].at[pl.ds(i * _HIST, _HIST)],
                        out_e.at[bbase + i], w_sem[b]).wait()

            @plsc.parallel_loop(0, _CL // _L, unroll=_CL // _L)
            def _(grp):
                ids16 = ids_v[pl.ds(c * _CL + grp * _L, _L)]
                cl16 = plsc.load_gather(elab_v, [ids16])
                bases = [cl16[l] * _EMBED_DIM for l in range(_L)]
                nk = _EMBED_DIM // _L
                vals = [etab_v[pl.ds(bases[l], _L)] for l in range(_L)]
                for k in range(nk):
                    cur = vals
                    if k + 1 < nk:
                        vals = [etab_v[pl.ds(bases[l] + (k + 1) * _L, _L)]
                                for l in range(_L)]
                    for l in range(_L):
                        rows3_v[b][grp * _L + l, pl.ds(k * _L, _L)] = cur[l]

            for i in range(_WB):
                pltpu.async_copy(
                    rows3_v[b].at[pl.ds(i * _HIST, _HIST)],
                    out_e.at[bbase + c * _WB + i], w_sem[b])
        return carry

    lax.fori_loop(0, _E_GROUPS, _egroup, 0)
    for b in range(2):
        for i in range(_WB):
            pltpu.make_async_copy(
                rows3_v[b].at[pl.ds(i * _HIST, _HIST)],
                out_e.at[bbase + i], w_sem[b]).wait()

    for c in range(_R_CHUNKS):
        pltpu.make_async_copy(rtab.at[rcl_v[c]], rbuf_v[c], r_sem[c]).wait()
        pltpu.async_copy(
            rbuf_v[c], out_r.at[pl.ds(rbase + c * _RCH, _RCH)], r_sem[c])
    for c in range(_R_CHUNKS):
        pltpu.make_async_copy(
            rbuf_v[c], out_r.at[pl.ds(rbase, _RCH)], r_sem[c]).wait()


def kernel(entity_ids, relation_ids, entity_cluster_labels,
           relation_cluster_labels, entity_table, relation_table):
    return _cluster_embed_sc(
        entity_ids.reshape(-1),
        relation_ids,
        entity_cluster_labels,
        relation_cluster_labels,
        entity_table.reshape(-1),
        relation_table,
    )
